# baseline (speedup 1.0000x reference)
import functools

import jax
import jax.numpy as jnp
from jax import lax
from jax.experimental import pallas as pl
from jax.experimental.pallas import tpu as pltpu
from jax.experimental.pallas import tpu_sc as plsc

N = 10000
E = 320000
D = 128

NC = 2
NS = 16
CHB = 128
NCH = 80
BK = 16
EP = NC * NS * NCH * CHB
NP = 10240
RPT = NP // NS
STRIP = 128
F = 128

_mesh = plsc.VectorSubcoreMesh(core_axis_name="c", subcore_axis_name="s")


HR = 40
NPASS = NP // (HR * 128)


@functools.partial(
    pl.kernel,
    out_type=jax.ShapeDtypeStruct((NC, NPASS * HR, 128), jnp.float32),
    mesh=_mesh,
    scratch_types=[
        pltpu.VMEM((NCH, CHB), jnp.int32),
        pltpu.VMEM((16 * HR * 128,), jnp.float32),
        pltpu.VMEM((HR, 128), jnp.float32),
        pltpu.VMEM((16, 128), jnp.float32),
        pltpu.VMEM((16, 128), jnp.float32),
        pltpu.VMEM_SHARED((NS, NPASS * HR, 128), jnp.float32),
    ],
    compiler_params=pltpu.CompilerParams(needs_layout_passes=False),
)
def _sc_degree(dst_hbm, zflat_hbm, out_hbm, dstv, hist, deg2, accv, ldv, degsh):
    c = lax.axis_index("c")
    s = lax.axis_index("s")
    pltpu.sync_copy(dst_hbm.at[c, s], dstv)
    lane = jax.lax.iota(jnp.int32, 16)
    ones = jnp.full((16,), 1.0, jnp.float32)
    for p in range(NPASS):
        base = p * HR * 128
        pltpu.sync_copy(zflat_hbm, hist)

        def chunk(i, carry):
            def sub(j, carry2):
                d = dstv[i, pl.ds(j * 16, 16)]
                m = (d >= base) & (d < base + HR * 128)
                idx = (lane * HR + ((d - base) >> 7)) * 128 + (d & 127)
                plsc.addupdate_scatter(hist, (idx,), ones, mask=m)
                return carry2
            return lax.fori_loop(0, CHB // 16, sub, carry)

        lax.fori_loop(0, NCH, chunk, 0)

        def fold(r, carry):
            for g in range(128 // 16):
                acc = hist[pl.ds(r * 128 + g * 16, 16)]
                for l in range(1, 16):
                    acc = acc + hist[pl.ds((l * HR + r) * 128 + g * 16, 16)]
                deg2[r, pl.ds(g * 16, 16)] = acc
            return carry

        lax.fori_loop(0, HR, fold, 0)
        pltpu.sync_copy(deg2, degsh.at[s, pl.ds(p * HR, HR)])
    plsc.subcore_barrier()

    @pl.when(s < (NPASS * HR) // 16)
    def _():
        gbase = s * 16
        pltpu.sync_copy(degsh.at[0, pl.ds(gbase, 16)], accv)
        for k in range(1, NS):
            pltpu.sync_copy(degsh.at[k, pl.ds(gbase, 16)], ldv)

            def addrow(r, carry):
                for g in range(8):
                    accv[r, pl.ds(g * 16, 16)] = (
                        accv[r, pl.ds(g * 16, 16)] + ldv[r, pl.ds(g * 16, 16)])
                return carry

            lax.fori_loop(0, 16, addrow, 0)
        pltpu.sync_copy(accv, out_hbm.at[c, pl.ds(gbase, 16)])


@functools.partial(
    pl.kernel,
    out_type=jax.ShapeDtypeStruct((NC, NP, F), jnp.float32),
    mesh=_mesh,
    scratch_types=[
        pltpu.VMEM((BK, CHB), jnp.int32),
        pltpu.VMEM((BK, CHB), jnp.int32),
        pltpu.VMEM((CHB, F), jnp.float32),
        pltpu.VMEM((CHB, F), jnp.float32),
        pltpu.VMEM_SHARED((NP, F), jnp.float32),
        pltpu.SemaphoreType.DMA,
        pltpu.SemaphoreType.DMA,
        pltpu.SemaphoreType.DMA,
        pltpu.SemaphoreType.DMA,
    ],
)
def _sc_segsum(src_hbm, dst_hbm, hp_hbm, out_hbm,
               srcv, dstv, rows0, rows1, agg_sh, g0, g1, s0, s1):
    c = lax.axis_index("c")
    s = lax.axis_index("s")
    base = s * RPT
    zv = jnp.zeros((16,), jnp.float32)

    def zero(k, carry):
        rows0[pl.ds(k * 16, 16), :] = jnp.zeros((16, F), jnp.float32)
        return carry

    lax.fori_loop(0, CHB // 16, zero, 0)
    for k in range(RPT // STRIP):
        pltpu.sync_copy(rows0, agg_sh.at[pl.ds(base + k * STRIP, STRIP)])
    plsc.subcore_barrier()

    for b in range(NCH // BK):
        pltpu.sync_copy(src_hbm.at[c, s, pl.ds(b * BK, BK)], srcv)
        pltpu.sync_copy(dst_hbm.at[c, s, pl.ds(b * BK, BK)], dstv)
        pltpu.async_copy(hp_hbm.at[srcv.at[0]], rows0, g0)

        def pair(k, carry):
            c0 = 2 * k

            @pl.when(k > 0)
            def _():
                pltpu.make_async_copy(rows1, agg_sh.at[dstv.at[c0 - 1]], s1).wait()

            pltpu.async_copy(hp_hbm.at[srcv.at[c0 + 1]], rows1, g1)
            pltpu.make_async_copy(hp_hbm.at[srcv.at[c0]], rows0, g0).wait()
            pltpu.async_copy(rows0, agg_sh.at[dstv.at[c0]], s0, add=True)
            pltpu.make_async_copy(rows0, agg_sh.at[dstv.at[c0]], s0).wait()

            @pl.when(k < BK // 2 - 1)
            def _():
                pltpu.async_copy(hp_hbm.at[srcv.at[c0 + 2]], rows0, g0)

            pltpu.make_async_copy(hp_hbm.at[srcv.at[c0 + 1]], rows1, g1).wait()
            pltpu.async_copy(rows1, agg_sh.at[dstv.at[c0 + 1]], s1, add=True)
            return carry

        lax.fori_loop(0, BK // 2, pair, 0)
        pltpu.make_async_copy(rows1, agg_sh.at[dstv.at[BK - 1]], s1).wait()
    plsc.subcore_barrier()
    for k in range(RPT // STRIP):
        off = base + k * STRIP
        buf = rows0 if k % 2 == 0 else rows1
        pltpu.sync_copy(agg_sh.at[pl.ds(off, STRIP)], buf)
        pltpu.sync_copy(buf, out_hbm.at[c, pl.ds(off, STRIP)])


def _tch_body(x, w1p, h_out):
    h_out[...] = jnp.dot(x[...], w1p[...], preferred_element_type=jnp.float32)


def _tc1_body(hist0, hist1, h, hp_out, dinv_out):
    dinv = lax.rsqrt(hist0[...] + hist1[...] + 1.0)
    hp_out[...] = h[...] * dinv
    dinv_out[...] = dinv


def _tc2_body(a0, a1, hp1, dinv, b1, w2p, hp2_out):
    dv = dinv[...]
    agg = a0[0:N, 0:64] + a1[0:N, 0:64] + hp1[0:N, 0:64]
    h1 = jnp.maximum(agg * dv + b1[...], 0.0)
    hp2_out[...] = jnp.dot(h1, w2p[...],
                           preferred_element_type=jnp.float32) * dv


def _tc3_body(a0, a1, hp2, dinv, b2, fw1, fb1, fw2, fb2, y_out):
    dv = dinv[...]
    agg = a0[0:N, 0:32] + a1[0:N, 0:32] + hp2[0:N, 0:32]
    h2 = jnp.maximum(agg * dv + b2[...], 0.0)
    h3 = jnp.maximum(
        jnp.dot(h2, fw1[...], preferred_element_type=jnp.float32) + fb1[...], 0.0)
    y_out[...] = jnp.dot(h3, fw2[...], preferred_element_type=jnp.float32) + fb2[...]


_tch = pl.pallas_call(
    _tch_body,
    out_shape=jax.ShapeDtypeStruct((N, F), jnp.float32),
)
_tc1 = pl.pallas_call(
    _tc1_body,
    out_shape=(jax.ShapeDtypeStruct((N, F), jnp.float32),
               jax.ShapeDtypeStruct((N, 1), jnp.float32)),
)
_tc2 = pl.pallas_call(
    _tc2_body,
    out_shape=jax.ShapeDtypeStruct((N, F), jnp.float32),
)
_tc3 = pl.pallas_call(
    _tc3_body,
    out_shape=jax.ShapeDtypeStruct((N, 1), jnp.float32),
)


def kernel(x, edge_index, W1, b1, W2, b2, fw1, fb1, fw2, fb2):
    pad = EP - E
    pad_ar = jnp.arange(pad, dtype=jnp.int32)
    src_p = jnp.concatenate([edge_index[0], pad_ar % N]).reshape(NC, NS, NCH, CHB)
    dst_p = jnp.concatenate([edge_index[1], N + (pad_ar % (NP - N))]
                            ).reshape(NC, NS, NCH, CHB)

    w1p = jnp.pad(W1, ((0, 0), (0, F - 64)))
    w2p = jnp.pad(W2, ((0, 0), (0, F - 32)))

    h = _tch(x, w1p)
    deg_part = _sc_degree(dst_p, jnp.zeros((16 * HR * 128,), jnp.float32))
    hist0 = jnp.reshape(deg_part[0], (NP, 1))[:N]
    hist1 = jnp.reshape(deg_part[1], (NP, 1))[:N]
    hp1, dinv = _tc1(hist0, hist1, h)
    agg1 = _sc_segsum(src_p, dst_p, hp1)
    hp2 = _tc2(agg1[0], agg1[1], hp1, dinv, b1[None, :], w2p)
    agg2 = _sc_segsum(src_p, dst_p, hp2)
    y = _tc3(agg2[0], agg2[1], hp2, dinv, b2[None, :],
             fw1, fb1[None, :], fw2, fb2[None, :])
    return jnp.squeeze(y, axis=-1)


# speedup vs baseline: 1.0826x; 1.0826x over previous
import functools

import jax
import jax.numpy as jnp
from jax import lax
from jax.experimental import pallas as pl
from jax.experimental.pallas import tpu as pltpu
from jax.experimental.pallas import tpu_sc as plsc

N = 10000
E = 320000
D = 128

NC = 2
NS = 16
CHB = 128
NCH = 80
BK = 40
EP = NC * NS * NCH * CHB
NP = 10240
RPT = NP // NS
STRIP = 128
F = 128

_mesh = plsc.VectorSubcoreMesh(core_axis_name="c", subcore_axis_name="s")


HR = 40
NPASS = NP // (HR * 128)


@functools.partial(
    pl.kernel,
    out_type=jax.ShapeDtypeStruct((NC, NS, NPASS * HR, 128), jnp.float32),
    mesh=_mesh,
    scratch_types=[
        pltpu.VMEM((NCH, CHB), jnp.int32),
        pltpu.VMEM((16 * HR * 128,), jnp.float32),
        pltpu.VMEM((HR, 128), jnp.float32),
    ],
    compiler_params=pltpu.CompilerParams(needs_layout_passes=False),
)
def _sc_degree(dst_hbm, zflat_hbm, out_hbm, dstv, hist, deg2):
    c = lax.axis_index("c")
    s = lax.axis_index("s")
    pltpu.sync_copy(dst_hbm.at[c, s], dstv)
    lane = jax.lax.iota(jnp.int32, 16)
    ones = jnp.full((16,), 1.0, jnp.float32)
    for p in range(NPASS):
        base = p * HR * 128
        pltpu.sync_copy(zflat_hbm, hist)

        def chunk(i, carry):
            def sub(j, carry2):
                d = dstv[i, pl.ds(j * 16, 16)]
                m = (d >= base) & (d < base + HR * 128)
                idx = (lane * HR + ((d - base) >> 7)) * 128 + (d & 127)
                plsc.addupdate_scatter(hist, (idx,), ones, mask=m)
                return carry2
            return lax.fori_loop(0, CHB // 16, sub, carry)

        lax.fori_loop(0, NCH, chunk, 0)

        def fold(r, carry):
            for g in range(128 // 16):
                acc = hist[pl.ds(r * 128 + g * 16, 16)]
                for l in range(1, 16):
                    acc = acc + hist[pl.ds((l * HR + r) * 128 + g * 16, 16)]
                deg2[r, pl.ds(g * 16, 16)] = acc
            return carry

        lax.fori_loop(0, HR, fold, 0)
        pltpu.sync_copy(deg2, out_hbm.at[c, s, pl.ds(p * HR, HR)])


@functools.partial(
    pl.kernel,
    out_type=jax.ShapeDtypeStruct((NC, NP, F), jnp.float32),
    mesh=_mesh,
    scratch_types=[
        pltpu.VMEM((BK, CHB), jnp.int32),
        pltpu.VMEM((BK, CHB), jnp.int32),
        pltpu.VMEM((CHB, F), jnp.float32),
        pltpu.VMEM((CHB, F), jnp.float32),
        pltpu.VMEM_SHARED((NP, F), jnp.float32),
        pltpu.SemaphoreType.DMA,
        pltpu.SemaphoreType.DMA,
        pltpu.SemaphoreType.DMA,
        pltpu.SemaphoreType.DMA,
    ],
)
def _sc_segsum(src_hbm, dst_hbm, hp_hbm, out_hbm,
               srcv, dstv, rows0, rows1, agg_sh, g0, g1, s0, s1):
    c = lax.axis_index("c")
    s = lax.axis_index("s")
    base = s * RPT
    zv = jnp.zeros((16,), jnp.float32)

    def zero(k, carry):
        rows0[pl.ds(k * 16, 16), :] = jnp.zeros((16, F), jnp.float32)
        return carry

    lax.fori_loop(0, CHB // 16, zero, 0)
    for k in range(RPT // STRIP):
        pltpu.sync_copy(rows0, agg_sh.at[pl.ds(base + k * STRIP, STRIP)])
    plsc.subcore_barrier()

    for b in range(NCH // BK):
        pltpu.sync_copy(src_hbm.at[c, s, pl.ds(b * BK, BK)], srcv)
        pltpu.sync_copy(dst_hbm.at[c, s, pl.ds(b * BK, BK)], dstv)
        pltpu.async_copy(hp_hbm.at[srcv.at[0]], rows0, g0)

        def pair(k, carry):
            c0 = 2 * k

            @pl.when(k > 0)
            def _():
                pltpu.make_async_copy(rows1, agg_sh.at[dstv.at[c0 - 1]], s1).wait()

            pltpu.async_copy(hp_hbm.at[srcv.at[c0 + 1]], rows1, g1)
            pltpu.make_async_copy(hp_hbm.at[srcv.at[c0]], rows0, g0).wait()
            pltpu.async_copy(rows0, agg_sh.at[dstv.at[c0]], s0, add=True)
            pltpu.make_async_copy(rows0, agg_sh.at[dstv.at[c0]], s0).wait()

            @pl.when(k < BK // 2 - 1)
            def _():
                pltpu.async_copy(hp_hbm.at[srcv.at[c0 + 2]], rows0, g0)

            pltpu.make_async_copy(hp_hbm.at[srcv.at[c0 + 1]], rows1, g1).wait()
            pltpu.async_copy(rows1, agg_sh.at[dstv.at[c0 + 1]], s1, add=True)
            return carry

        lax.fori_loop(0, BK // 2, pair, 0)
        pltpu.make_async_copy(rows1, agg_sh.at[dstv.at[BK - 1]], s1).wait()
    plsc.subcore_barrier()
    for k in range(RPT // STRIP):
        off = base + k * STRIP
        buf = rows0 if k % 2 == 0 else rows1
        pltpu.sync_copy(agg_sh.at[pl.ds(off, STRIP)], buf)
        pltpu.sync_copy(buf, out_hbm.at[c, pl.ds(off, STRIP)])


def _tc0_body(deg, dinv2d_out):
    hist = jnp.sum(deg[...], axis=(0, 1))
    dinv2d_out[...] = lax.rsqrt(hist + 1.0)


def _tc1_body(dinv, x, w1p, hp_out):
    h = jnp.dot(x[...], w1p[...], preferred_element_type=jnp.float32)
    hp_out[...] = h * dinv[...]


def _tc2_body(a0, a1, hp1, dinv, b1, w2p, hp2_out):
    dv = dinv[...]
    agg = a0[0:N, 0:64] + a1[0:N, 0:64] + hp1[0:N, 0:64]
    h1 = jnp.maximum(agg * dv + b1[...], 0.0)
    hp2_out[...] = jnp.dot(h1, w2p[...],
                           preferred_element_type=jnp.float32) * dv


def _tc3_body(a0, a1, hp2, dinv, b2, fw1, fb1, fw2, fb2, y_out):
    dv = dinv[...]
    agg = a0[0:N, 0:32] + a1[0:N, 0:32] + hp2[0:N, 0:32]
    h2 = jnp.maximum(agg * dv + b2[...], 0.0)
    h3 = jnp.maximum(
        jnp.dot(h2, fw1[...], preferred_element_type=jnp.float32) + fb1[...], 0.0)
    y_out[...] = jnp.dot(h3, fw2[...], preferred_element_type=jnp.float32) + fb2[...]


_tc0 = pl.pallas_call(
    _tc0_body,
    out_shape=jax.ShapeDtypeStruct((NPASS * HR, 128), jnp.float32),
)
_tc1 = pl.pallas_call(
    _tc1_body,
    out_shape=jax.ShapeDtypeStruct((N, F), jnp.float32),
)
_tc2 = pl.pallas_call(
    _tc2_body,
    out_shape=jax.ShapeDtypeStruct((N, F), jnp.float32),
)
_tc3 = pl.pallas_call(
    _tc3_body,
    out_shape=jax.ShapeDtypeStruct((N, 1), jnp.float32),
)


def kernel(x, edge_index, W1, b1, W2, b2, fw1, fb1, fw2, fb2):
    pad = EP - E
    pad_ar = jnp.arange(pad, dtype=jnp.int32)
    src_p = jnp.concatenate([edge_index[0], pad_ar % N]).reshape(NC, NS, NCH, CHB)
    dst_p = jnp.concatenate([edge_index[1], N + (pad_ar % (NP - N))]
                            ).reshape(NC, NS, NCH, CHB)

    w1p = jnp.pad(W1, ((0, 0), (0, F - 64)))
    w2p = jnp.pad(W2, ((0, 0), (0, F - 32)))

    deg_slabs = _sc_degree(dst_p, jnp.zeros((16 * HR * 128,), jnp.float32))
    dinv2d = _tc0(deg_slabs)
    dinv = jnp.reshape(dinv2d, (NP, 1))[:N]
    hp1 = _tc1(dinv, x, w1p)
    agg1 = _sc_segsum(src_p, dst_p, hp1)
    hp2 = _tc2(agg1[0], agg1[1], hp1, dinv, b1[None, :], w2p)
    agg2 = _sc_segsum(src_p, dst_p, hp2)
    y = _tc3(agg2[0], agg2[1], hp2, dinv, b2[None, :],
             fw1, fb1[None, :], fw2, fb2[None, :])
    return jnp.squeeze(y, axis=-1)
